# Initial kernel scaffold; baseline (speedup 1.0000x reference)
import jax
import jax.numpy as jnp
from jax.experimental import pallas as pl


def kernel(t, hierarchy_labels, masking_rates):
    raise NotImplementedError("write your pallas kernel here")


# speedup vs baseline: 172.0199x; 172.0199x over previous
import functools

import jax
import jax.numpy as jnp
from jax import lax
from jax.experimental import pallas as pl
from jax.experimental.pallas import tpu as pltpu
from jax.experimental.pallas import tpu_sc as plsc

NUM_LEVELS = 4
TABLE_W = 1001
BATCH = 4096
SEQ_L = 200

NUM_WORKERS = 32
ROWS_PER_W = BATCH // NUM_WORKERS
ELEMS_PER_W = ROWS_PER_W * SEQ_L
NVREG = ELEMS_PER_W // 16

_MESH = plsc.VectorSubcoreMesh(core_axis_name="c", subcore_axis_name="s")


@functools.partial(
    pl.kernel,
    out_type=jax.ShapeDtypeStruct((BATCH * SEQ_L,), jnp.float32),
    mesh=_MESH,
    scratch_types=[
        pltpu.VMEM((NUM_LEVELS, TABLE_W), jnp.float32),
        pltpu.VMEM((ROWS_PER_W,), jnp.int32),
        pltpu.VMEM((ELEMS_PER_W,), jnp.int32),
        pltpu.VMEM((ROWS_PER_W * NUM_LEVELS,), jnp.float32),
        pltpu.VMEM((ELEMS_PER_W,), jnp.float32),
    ],
    compiler_params=pltpu.CompilerParams(needs_layout_passes=False),
)
def _sc_gather(t_hbm, labels_hbm, table_hbm, out_hbm,
               table_v, t_v, lbl_v, rates_v, out_v):
    wid = lax.axis_index("s") * 2 + lax.axis_index("c")
    row0 = wid * ROWS_PER_W
    e0 = wid * ELEMS_PER_W

    pltpu.sync_copy(table_hbm, table_v)
    pltpu.sync_copy(t_hbm.at[pl.ds(row0, ROWS_PER_W)], t_v)
    pltpu.sync_copy(labels_hbm.at[pl.ds(e0, ELEMS_PER_W)], lbl_v)

    lane = lax.iota(jnp.int32, 16)

    def stage1(j, _):
        p = j * 16 + lane
        r = lax.shift_right_logical(p, 2)
        k = lax.bitwise_and(p, 3)
        tb = plsc.load_gather(t_v, [r])
        val = plsc.load_gather(table_v, [k, tb])
        rates_v[pl.ds(pl.multiple_of(j * 16, 16), 16)] = val
        return 0

    lax.fori_loop(0, (ROWS_PER_W * NUM_LEVELS) // 16, stage1, 0)

    def stage2(i, _):
        base = pl.multiple_of(i * 16, 16)
        lbl = lbl_v[pl.ds(base, 16)]
        p = i * 16 + lane
        r = lax.shift_right_logical(p * 41944, 23)
        idx = lax.shift_left(r, 2) + lbl
        out_v[pl.ds(base, 16)] = plsc.load_gather(rates_v, [idx])
        return 0

    lax.fori_loop(0, NVREG, stage2, 0)

    pltpu.sync_copy(out_v, out_hbm.at[pl.ds(e0, ELEMS_PER_W)])


def kernel(t, hierarchy_labels, masking_rates):
    t = t.astype(jnp.int32)
    labels_flat = hierarchy_labels.astype(jnp.int32).reshape(-1)
    out = _sc_gather(t, labels_flat, masking_rates.astype(jnp.float32))
    return out.reshape(BATCH, SEQ_L)


# speedup vs baseline: 230.5083x; 1.3400x over previous
import functools

import jax
import jax.numpy as jnp
from jax import lax
from jax.experimental import pallas as pl
from jax.experimental.pallas import tpu as pltpu
from jax.experimental.pallas import tpu_sc as plsc

NUM_LEVELS = 4
TABLE_W = 1001
BATCH = 4096
SEQ_L = 200

NUM_WORKERS = 32
ROWS_PER_W = BATCH // NUM_WORKERS
ELEMS_PER_W = ROWS_PER_W * SEQ_L
NVREG = ELEMS_PER_W // 16

_MESH = plsc.VectorSubcoreMesh(core_axis_name="c", subcore_axis_name="s")


@functools.partial(
    pl.kernel,
    out_type=jax.ShapeDtypeStruct((BATCH * SEQ_L,), jnp.float32),
    mesh=_MESH,
    scratch_types=[
        pltpu.VMEM((NUM_LEVELS, TABLE_W), jnp.float32),
        pltpu.VMEM((ROWS_PER_W,), jnp.int32),
        pltpu.VMEM((ELEMS_PER_W,), jnp.int32),
        pltpu.VMEM((ROWS_PER_W * NUM_LEVELS,), jnp.float32),
        pltpu.VMEM((ELEMS_PER_W,), jnp.float32),
    ],
    compiler_params=pltpu.CompilerParams(needs_layout_passes=False),
)
def _sc_gather(t_hbm, labels_hbm, table_hbm, out_hbm,
               table_v, t_v, lbl_v, rates_v, out_v):
    wid = lax.axis_index("s") * 2 + lax.axis_index("c")
    row0 = wid * ROWS_PER_W
    e0 = wid * ELEMS_PER_W

    pltpu.sync_copy(table_hbm, table_v)
    pltpu.sync_copy(t_hbm.at[pl.ds(row0, ROWS_PER_W)], t_v)
    pltpu.sync_copy(labels_hbm.at[pl.ds(e0, ELEMS_PER_W)], lbl_v)

    lane = lax.iota(jnp.int32, 16)

    @plsc.parallel_loop(0, ROWS_PER_W * NUM_LEVELS, 16, unroll=2)
    def stage1(p0):
        base = pl.multiple_of(p0, 16)
        p = p0 + lane
        r = lax.shift_right_logical(p, 2)
        k = lax.bitwise_and(p, 3)
        tb = plsc.load_gather(t_v, [r])
        rates_v[pl.ds(base, 16)] = plsc.load_gather(table_v, [k, tb])

    @plsc.parallel_loop(0, ELEMS_PER_W, 16, unroll=8)
    def stage2(p0):
        base = pl.multiple_of(p0, 16)
        lbl = lbl_v[pl.ds(base, 16)]
        p = p0 + lane
        r = lax.shift_right_logical(p * 41944, 23)
        idx = lax.shift_left(r, 2) + lbl
        out_v[pl.ds(base, 16)] = plsc.load_gather(rates_v, [idx])

    pltpu.sync_copy(out_v, out_hbm.at[pl.ds(e0, ELEMS_PER_W)])


def kernel(t, hierarchy_labels, masking_rates):
    t = t.astype(jnp.int32)
    labels_flat = hierarchy_labels.astype(jnp.int32).reshape(-1)
    out = _sc_gather(t, labels_flat, masking_rates.astype(jnp.float32))
    return out.reshape(BATCH, SEQ_L)


# speedup vs baseline: 264.1489x; 1.1459x over previous
import functools

import jax
import jax.numpy as jnp
from jax import lax
from jax.experimental import pallas as pl
from jax.experimental.pallas import tpu as pltpu
from jax.experimental.pallas import tpu_sc as plsc

NUM_LEVELS = 4
TABLE_W = 1001
BATCH = 4096
SEQ_L = 200

NUM_WORKERS = 32
ROWS_PER_W = BATCH // NUM_WORKERS
ELEMS_PER_W = ROWS_PER_W * SEQ_L

_MESH = plsc.VectorSubcoreMesh(core_axis_name="c", subcore_axis_name="s")


@functools.partial(
    pl.kernel,
    out_type=jax.ShapeDtypeStruct((BATCH, SEQ_L), jnp.float32),
    mesh=_MESH,
    scratch_types=[
        pltpu.VMEM((NUM_LEVELS, TABLE_W), jnp.float32),
        pltpu.VMEM((ROWS_PER_W,), jnp.int32),
        pltpu.VMEM((ROWS_PER_W, SEQ_L), jnp.int32),
        pltpu.VMEM((ROWS_PER_W * NUM_LEVELS,), jnp.float32),
        pltpu.VMEM((ROWS_PER_W, SEQ_L), jnp.float32),
    ],
    compiler_params=pltpu.CompilerParams(needs_layout_passes=False),
)
def _sc_gather(t_hbm, labels_hbm, table_hbm, out_hbm,
               table_v, t_v, lbl_v, rates_v, out_v):
    wid = lax.axis_index("s") * 2 + lax.axis_index("c")
    row0 = wid * ROWS_PER_W

    pltpu.sync_copy(table_hbm, table_v)
    pltpu.sync_copy(t_hbm.at[pl.ds(row0, ROWS_PER_W)], t_v)
    pltpu.sync_copy(labels_hbm.at[pl.ds(row0, ROWS_PER_W)], lbl_v)

    lane = lax.iota(jnp.int32, 16)

    @plsc.parallel_loop(0, ROWS_PER_W * NUM_LEVELS, 16, unroll=2)
    def stage1(p0):
        base = pl.multiple_of(p0, 16)
        p = p0 + lane
        r = lax.shift_right_logical(p, 2)
        k = lax.bitwise_and(p, 3)
        tb = plsc.load_gather(t_v, [r])
        rates_v[pl.ds(base, 16)] = plsc.load_gather(table_v, [k, tb])

    @plsc.parallel_loop(0, ELEMS_PER_W, 16, unroll=8)
    def stage2(p0):
        p = p0 + lane
        r = lax.shift_right_logical(p * 41944, 23)
        c = p - r * 200
        lbl = plsc.load_gather(lbl_v, [r, c])
        val = plsc.load_gather(rates_v, [lax.shift_left(r, 2) + lbl])
        plsc.store_scatter(out_v, [r, c], val)

    pltpu.sync_copy(out_v, out_hbm.at[pl.ds(row0, ROWS_PER_W)])


def kernel(t, hierarchy_labels, masking_rates):
    return _sc_gather(t.astype(jnp.int32),
                      hierarchy_labels.astype(jnp.int32),
                      masking_rates.astype(jnp.float32))


# speedup vs baseline: 271.4261x; 1.0275x over previous
import functools

import jax
import jax.numpy as jnp
from jax import lax
from jax.experimental import pallas as pl
from jax.experimental.pallas import tpu as pltpu
from jax.experimental.pallas import tpu_sc as plsc

NUM_LEVELS = 4
TABLE_W = 1001
BATCH = 4096
SEQ_L = 200

NUM_WORKERS = 32
ROWS_PER_W = BATCH // NUM_WORKERS
ELEMS_PER_W = ROWS_PER_W * SEQ_L

_MESH = plsc.VectorSubcoreMesh(core_axis_name="c", subcore_axis_name="s")


@functools.partial(
    pl.kernel,
    out_type=jax.ShapeDtypeStruct((BATCH, SEQ_L), jnp.float32),
    mesh=_MESH,
    scratch_types=[
        pltpu.VMEM((NUM_LEVELS, TABLE_W), jnp.float32),
        pltpu.VMEM((ROWS_PER_W,), jnp.int32),
        pltpu.VMEM((ROWS_PER_W, SEQ_L), jnp.int32),
        pltpu.VMEM((ROWS_PER_W * NUM_LEVELS,), jnp.float32),
        pltpu.VMEM((ROWS_PER_W, SEQ_L), jnp.float32),
    ],
    compiler_params=pltpu.CompilerParams(needs_layout_passes=False,
                                         use_tc_tiling_on_sc=True),
)
def _sc_gather(t_hbm, labels_hbm, table_hbm, out_hbm,
               table_v, t_v, lbl_v, rates_v, out_v):
    wid = lax.axis_index("s") * 2 + lax.axis_index("c")
    row0 = wid * ROWS_PER_W

    pltpu.sync_copy(table_hbm, table_v)
    pltpu.sync_copy(t_hbm.at[pl.ds(row0, ROWS_PER_W)], t_v)
    pltpu.sync_copy(labels_hbm.at[pl.ds(row0, ROWS_PER_W)], lbl_v)

    lane = lax.iota(jnp.int32, 16)

    @plsc.parallel_loop(0, ROWS_PER_W * NUM_LEVELS, 16, unroll=2)
    def stage1(p0):
        base = pl.multiple_of(p0, 16)
        p = p0 + lane
        r = lax.shift_right_logical(p, 2)
        k = lax.bitwise_and(p, 3)
        tb = plsc.load_gather(t_v, [r])
        rates_v[pl.ds(base, 16)] = plsc.load_gather(table_v, [k, tb])

    @plsc.parallel_loop(0, ELEMS_PER_W, 16, unroll=8)
    def stage2(p0):
        p = p0 + lane
        r = lax.shift_right_logical(p * 41944, 23)
        c = p - r * 200
        lbl = plsc.load_gather(lbl_v, [r, c])
        val = plsc.load_gather(rates_v, [lax.shift_left(r, 2) + lbl])
        plsc.store_scatter(out_v, [r, c], val)

    pltpu.sync_copy(out_v, out_hbm.at[pl.ds(row0, ROWS_PER_W)])


def kernel(t, hierarchy_labels, masking_rates):
    return _sc_gather(t.astype(jnp.int32),
                      hierarchy_labels.astype(jnp.int32),
                      masking_rates.astype(jnp.float32))


# speedup vs baseline: 295.0758x; 1.0871x over previous
import functools

import jax
import jax.numpy as jnp
from jax import lax
from jax.experimental import pallas as pl
from jax.experimental.pallas import tpu as pltpu
from jax.experimental.pallas import tpu_sc as plsc

NUM_LEVELS = 4
TABLE_W = 1001
BATCH = 4096
SEQ_L = 200

NUM_WORKERS = 32
ROWS_PER_W = BATCH // NUM_WORKERS
ELEMS_PER_W = ROWS_PER_W * SEQ_L

_MESH = plsc.VectorSubcoreMesh(core_axis_name="c", subcore_axis_name="s")


@functools.partial(
    pl.kernel,
    out_type=jax.ShapeDtypeStruct((BATCH, SEQ_L), jnp.float32),
    mesh=_MESH,
    scratch_types=[
        pltpu.VMEM((NUM_LEVELS, TABLE_W), jnp.float32),
        pltpu.VMEM((ROWS_PER_W,), jnp.int32),
        pltpu.VMEM((ROWS_PER_W, SEQ_L), jnp.int32),
        pltpu.VMEM((ROWS_PER_W * NUM_LEVELS,), jnp.float32),
        pltpu.VMEM((ROWS_PER_W, SEQ_L), jnp.float32),
    ],
    compiler_params=pltpu.CompilerParams(needs_layout_passes=False),
)
def _sc_gather(t_hbm, labels_hbm, table_hbm, out_hbm,
               table_v, t_v, lbl_v, rates_v, out_v):
    wid = lax.axis_index("s") * 2 + lax.axis_index("c")
    row0 = wid * ROWS_PER_W

    pltpu.sync_copy(table_hbm, table_v)
    pltpu.sync_copy(t_hbm.at[pl.ds(row0, ROWS_PER_W)], t_v)
    pltpu.sync_copy(labels_hbm.at[pl.ds(row0, ROWS_PER_W)], lbl_v)

    lane = lax.iota(jnp.int32, 16)

    @plsc.parallel_loop(0, ROWS_PER_W * NUM_LEVELS, 16, unroll=2)
    def stage1(p0):
        base = pl.multiple_of(p0, 16)
        p = p0 + lane
        r = lax.shift_right_logical(p, 2)
        k = lax.bitwise_and(p, 3)
        tb = plsc.load_gather(t_v, [r])
        rates_v[pl.ds(base, 16)] = plsc.load_gather(table_v, [k, tb])

    col_starts = tuple(range(0, SEQ_L - 16, 16)) + (SEQ_L - 16,)

    @plsc.parallel_loop(0, ROWS_PER_W, 1)
    def stage2(r):
        r4 = lax.shift_left(r, 2)
        for c0 in col_starts:
            lbl = lbl_v[r, pl.ds(c0, 16)]
            out_v[r, pl.ds(c0, 16)] = plsc.load_gather(rates_v, [r4 + lbl])

    pltpu.sync_copy(out_v, out_hbm.at[pl.ds(row0, ROWS_PER_W)])


def kernel(t, hierarchy_labels, masking_rates):
    return _sc_gather(t.astype(jnp.int32),
                      hierarchy_labels.astype(jnp.int32),
                      masking_rates.astype(jnp.float32))
